# baseline (speedup 1.0000x reference)
import functools
import jax
import jax.numpy as jnp
from jax import lax
from jax.experimental import pallas as pl
from jax.experimental.pallas import tpu as pltpu
from jax.experimental.pallas import tpu_sc as plsc

_NC = 2
_NS = 16
_L = 16
_NW = _NC * _NS

_M = 16384
_N = 4096
_G = 8
_SEG = _N // _L
_ROWS_PER_W = _M // _NW


def _sc_body(x_hbm, out_hbm, xin_v, yout_v):
    wid = lax.axis_index("s") * _NC + lax.axis_index("c")
    base = wid * _ROWS_PER_W * _N
    seg = lax.iota(jnp.int32, _L) * _SEG

    def group_body(g, carry_unused):
        el0 = base + g * (_G * _N)
        pltpu.sync_copy(x_hbm.at[pl.ds(el0, _G * _N)], xin_v)
        for r in range(_G):
            rbase = seg + r * _N

            def pass_a(i, acc):
                v = plsc.load_gather(xin_v, [rbase + i])
                return acc + v

            acc = lax.fori_loop(0, _SEG, pass_a, jnp.zeros((_L,), jnp.float32))
            offs = plsc.cumsum(acc) - acc

            def pass_b(i, carry):
                v = plsc.load_gather(xin_v, [rbase + i])
                carry = carry + v
                plsc.store_scatter(yout_v, [rbase + i], carry)
                return carry

            lax.fori_loop(0, _SEG, pass_b, offs)
        pltpu.sync_copy(yout_v, out_hbm.at[pl.ds(el0, _G * _N)])
        return carry_unused

    lax.fori_loop(0, _ROWS_PER_W // _G, group_body, 0)


def kernel(x):
    orig_dtype = x.dtype
    xf = x.astype(jnp.float32)
    B, S, N = xf.shape
    x2 = xf.reshape(B * S * N)
    mesh = plsc.VectorSubcoreMesh(core_axis_name="c", subcore_axis_name="s")
    sck = functools.partial(
        pl.kernel,
        mesh=mesh,
        out_type=jax.ShapeDtypeStruct((_M * _N,), jnp.float32),
        scratch_types=[
            pltpu.VMEM((_G * _N,), jnp.float32),
            pltpu.VMEM((_G * _N,), jnp.float32),
        ],
        compiler_params=pltpu.CompilerParams(needs_layout_passes=False),
    )(_sc_body)
    out = sck(x2)
    return out.reshape(B, S, N).astype(orig_dtype)


# speedup vs baseline: 1.4643x; 1.4643x over previous
import functools
import jax
import jax.numpy as jnp
from jax import lax
from jax.experimental import pallas as pl
from jax.experimental.pallas import tpu as pltpu
from jax.experimental.pallas import tpu_sc as plsc

_NC = 2
_NS = 16
_L = 16
_NW = _NC * _NS

_M = 16384
_N = 4096
_G = 8
_SEG = _N // _L
_ROWS_PER_W = _M // _NW
_UNROLL = 16


def _sc_body(x_hbm, out_hbm, xin_v, yout_v):
    wid = lax.axis_index("s") * _NC + lax.axis_index("c")
    base = wid * _ROWS_PER_W * _N
    seg = lax.iota(jnp.int32, _L) * _SEG

    def group_body(g, carry_unused):
        el0 = base + g * (_G * _N)
        pltpu.sync_copy(x_hbm.at[pl.ds(el0, _G * _N)], xin_v)
        for r in range(_G):
            rbase = seg + r * _N

            def pass_a(ii, acc):
                s = rbase + ii * _UNROLL
                vs = [plsc.load_gather(xin_v, [s + k]) for k in range(_UNROLL)]
                while len(vs) > 1:
                    vs = [vs[i] + vs[i + 1] for i in range(0, len(vs) - 1, 2)] \
                        + ([vs[-1]] if len(vs) % 2 else [])
                return acc + vs[0]

            acc = lax.fori_loop(0, _SEG // _UNROLL, pass_a,
                                jnp.zeros((_L,), jnp.float32))
            offs = plsc.cumsum(acc) - acc

            def pass_b(ii, carry):
                s = rbase + ii * _UNROLL
                vs = [plsc.load_gather(xin_v, [s + k]) for k in range(_UNROLL)]
                for k in range(_UNROLL):
                    carry = carry + vs[k]
                    plsc.store_scatter(yout_v, [s + k], carry)
                return carry

            lax.fori_loop(0, _SEG // _UNROLL, pass_b, offs)
        pltpu.sync_copy(yout_v, out_hbm.at[pl.ds(el0, _G * _N)])
        return carry_unused

    lax.fori_loop(0, _ROWS_PER_W // _G, group_body, 0)


def kernel(x):
    orig_dtype = x.dtype
    xf = x.astype(jnp.float32)
    B, S, N = xf.shape
    x2 = xf.reshape(B * S * N)
    mesh = plsc.VectorSubcoreMesh(core_axis_name="c", subcore_axis_name="s")
    sck = functools.partial(
        pl.kernel,
        mesh=mesh,
        out_type=jax.ShapeDtypeStruct((_M * _N,), jnp.float32),
        scratch_types=[
            pltpu.VMEM((_G * _N,), jnp.float32),
            pltpu.VMEM((_G * _N,), jnp.float32),
        ],
        compiler_params=pltpu.CompilerParams(needs_layout_passes=False),
    )(_sc_body)
    out = sck(x2)
    return out.reshape(B, S, N).astype(orig_dtype)


# speedup vs baseline: 27.9118x; 19.0620x over previous
import jax
import jax.numpy as jnp
from jax.experimental import pallas as pl
from jax.experimental.pallas import tpu as pltpu

_R = 512
_C = 128
_N = 4096


def _body(x_ref, u_ref, o_ref):
    u = u_ref[...]
    carry = jnp.zeros((_R, 1), jnp.float32)
    for c in range(_N // _C):
        blk = x_ref[:, c * _C:(c + 1) * _C]
        y = jax.lax.dot_general(
            blk, u, (((1,), (0,)), ((), ())),
            preferred_element_type=jnp.float32,
            precision=jax.lax.Precision.DEFAULT,
        )
        y = y + carry
        o_ref[:, c * _C:(c + 1) * _C] = y
        carry = y[:, _C - 1:_C]


def kernel(x):
    orig_dtype = x.dtype
    xf = x.astype(jnp.float32)
    B, S, N = xf.shape
    M = B * S
    x2 = xf.reshape(M, N)
    U = jnp.triu(jnp.ones((_C, _C), jnp.float32))
    grid = (M // _R,)
    out = pl.pallas_call(
        _body,
        grid=grid,
        in_specs=[
            pl.BlockSpec((_R, N), lambda i: (i, 0)),
            pl.BlockSpec((_C, _C), lambda i: (0, 0)),
        ],
        out_specs=pl.BlockSpec((_R, N), lambda i: (i, 0)),
        out_shape=jax.ShapeDtypeStruct((M, N), jnp.float32),
    )(x2, U)
    return out.reshape(B, S, N).astype(orig_dtype)
